# baseline (speedup 1.0000x reference)
import functools

import jax
import jax.numpy as jnp
from jax import lax
from jax.experimental import pallas as pl
from jax.experimental.pallas import tpu as pltpu
from jax.experimental.pallas import tpu_sc as plsc

_NW = 32
_RPS = 2
_SPLIT = 128


@jax.jit
def kernel(input_ids, weight):
    B, T = input_ids.shape
    V, D = weight.shape
    rows_w = B // _NW
    n_grp = rows_w // _RPS
    assert B % _NW == 0 and rows_w % _RPS == 0 and n_grp % 2 == 0

    ids = input_ids.astype(jnp.int32)

    mesh = plsc.VectorSubcoreMesh(core_axis_name="c", subcore_axis_name="s")

    @functools.partial(
        pl.kernel,
        mesh=mesh,
        compiler_params=pltpu.CompilerParams(use_tc_tiling_on_sc=False),
        out_type=jax.ShapeDtypeStruct((B, T, D), jnp.float32),
        scratch_types=[
            pltpu.VMEM((rows_w, T), jnp.int32),
            pltpu.VMEM((_RPS, T, D), jnp.float32),
            pltpu.VMEM((_RPS, T, D), jnp.float32),
            pltpu.SemaphoreType.DMA,
            pltpu.SemaphoreType.DMA,
            pltpu.SemaphoreType.DMA,
            pltpu.SemaphoreType.DMA,
        ],
    )
    def emb(ids_hbm, w_hbm, out_hbm, idx_v, slab0, slab1, g0, g1, o0, o1):
        wid = lax.axis_index("s") * 2 + lax.axis_index("c")
        rbase = wid * rows_w
        slabs = (slab0, slab1)
        gsem = (g0, g1)
        osem = (o0, o1)

        pltpu.sync_copy(ids_hbm.at[pl.ds(rbase, rows_w)], idx_v)

        def gathers(grp, p):
            for i in range(_RPS):
                r = grp * _RPS + i
                for off, ln in ((0, _SPLIT), (_SPLIT, T - _SPLIT)):
                    yield (
                        w_hbm.at[idx_v.at[r, pl.ds(off, ln)]],
                        slabs[p].at[i, pl.ds(off, ln)],
                        gsem[p],
                    )

        def fire(grp, p):
            for src, dst, sem in gathers(grp, p):
                pltpu.async_copy(src, dst, sem)

        def drain_and_out(grp, p):
            for src, dst, sem in gathers(grp, p):
                pltpu.make_async_copy(src, dst, sem).wait()
            return pltpu.async_copy(
                slabs[p], out_hbm.at[pl.ds(rbase + grp * _RPS, _RPS)], osem[p]
            )

        fire(0, 0)
        fire(1, 1)

        def body(g2, carry):
            for p in range(2):
                grp = g2 * 2 + p
                out_cp = drain_and_out(grp, p)
                out_cp.wait()
                fire(grp + 2, p)
            return carry

        lax.fori_loop(0, (n_grp - 2) // 2, body, 0)

        drain_and_out(n_grp - 2, 0).wait()
        drain_and_out(n_grp - 1, 1).wait()

    return emb(ids, weight)


# speedup vs baseline: 1.0011x; 1.0011x over previous
import functools

import jax
import jax.numpy as jnp
from jax import lax
from jax.experimental import pallas as pl
from jax.experimental.pallas import tpu as pltpu
from jax.experimental.pallas import tpu_sc as plsc

_CH = 512
_NW = 32


@jax.jit
def kernel(input_ids, weight):
    B, T = input_ids.shape
    V, D = weight.shape
    n = B * T
    n_chunks = n // _CH
    n_grp = n_chunks // _NW
    assert n % _CH == 0 and n_chunks % _NW == 0 and n_grp % 2 == 0

    ids = input_ids.reshape(n_chunks, _CH).astype(jnp.int32)

    mesh = plsc.VectorSubcoreMesh(core_axis_name="c", subcore_axis_name="s")

    @functools.partial(
        pl.kernel,
        mesh=mesh,
        compiler_params=pltpu.CompilerParams(use_tc_tiling_on_sc=False),
        out_type=jax.ShapeDtypeStruct((n_chunks, _CH, D), jnp.float32),
        scratch_types=[
            pltpu.VMEM((n_grp, _CH), jnp.int32),
            pltpu.VMEM((_CH, D), jnp.float32),
            pltpu.VMEM((_CH, D), jnp.float32),
            pltpu.SemaphoreType.DMA,
            pltpu.SemaphoreType.DMA,
            pltpu.SemaphoreType.DMA,
            pltpu.SemaphoreType.DMA,
        ],
    )
    def emb(ids_hbm, w_hbm, out_hbm, idx_v, slab0, slab1, g0, g1, o0, o1):
        wid = lax.axis_index("s") * 2 + lax.axis_index("c")
        cbase = wid * n_grp
        slabs = (slab0, slab1)
        gsem = (g0, g1)
        osem = (o0, o1)

        pltpu.sync_copy(ids_hbm.at[pl.ds(cbase, n_grp)], idx_v)

        def fire(grp, p):
            pltpu.async_copy(w_hbm.at[idx_v.at[grp]], slabs[p], gsem[p])

        def drain_and_out(grp, p):
            pltpu.make_async_copy(
                w_hbm.at[idx_v.at[grp]], slabs[p], gsem[p]
            ).wait()
            return pltpu.async_copy(slabs[p], out_hbm.at[cbase + grp], osem[p])

        fire(0, 0)
        fire(1, 1)

        def body(g2, carry):
            for p in range(2):
                grp = g2 * 2 + p
                out_cp = drain_and_out(grp, p)
                out_cp.wait()
                fire(grp + 2, p)
            return carry

        lax.fori_loop(0, (n_grp - 2) // 2, body, 0)

        drain_and_out(n_grp - 2, 0).wait()
        drain_and_out(n_grp - 1, 1).wait()

    out = emb(ids, weight)
    return out.reshape(B, T, D)


# speedup vs baseline: 1.0273x; 1.0262x over previous
import functools

import jax
import jax.numpy as jnp
from jax import lax
from jax.experimental import pallas as pl
from jax.experimental.pallas import tpu as pltpu
from jax.experimental.pallas import tpu_sc as plsc

_CH = 512
_NW = 32


@jax.jit
def kernel(input_ids, weight):
    B, T = input_ids.shape
    V, D = weight.shape
    n = B * T
    n_chunks = n // _CH
    n_grp = n_chunks // _NW
    assert n % _CH == 0 and n_chunks % _NW == 0 and n_grp % 2 == 0

    ids = input_ids.T.reshape(n_chunks, _CH).astype(jnp.int32)

    mesh = plsc.VectorSubcoreMesh(core_axis_name="c", subcore_axis_name="s")

    @functools.partial(
        pl.kernel,
        mesh=mesh,
        compiler_params=pltpu.CompilerParams(use_tc_tiling_on_sc=False),
        out_type=jax.ShapeDtypeStruct((n_chunks, _CH, D), jnp.float32),
        scratch_types=[
            pltpu.VMEM((n_grp, _CH), jnp.int32),
            pltpu.VMEM((_CH, D), jnp.float32),
            pltpu.VMEM((_CH, D), jnp.float32),
            pltpu.SemaphoreType.DMA,
            pltpu.SemaphoreType.DMA,
            pltpu.SemaphoreType.DMA,
            pltpu.SemaphoreType.DMA,
        ],
    )
    def emb(ids_hbm, w_hbm, out_hbm, idx_v, slab0, slab1, g0, g1, o0, o1):
        wid = lax.axis_index("s") * 2 + lax.axis_index("c")
        cbase = wid * n_grp
        slabs = (slab0, slab1)
        gsem = (g0, g1)
        osem = (o0, o1)

        pltpu.sync_copy(ids_hbm.at[pl.ds(cbase, n_grp)], idx_v)

        def fire(grp, p):
            pltpu.async_copy(w_hbm.at[idx_v.at[grp]], slabs[p], gsem[p])

        def drain_and_out(grp, p):
            pltpu.make_async_copy(
                w_hbm.at[idx_v.at[grp]], slabs[p], gsem[p]
            ).wait()
            return pltpu.async_copy(slabs[p], out_hbm.at[cbase + grp], osem[p])

        fire(0, 0)
        fire(1, 1)

        def body(g2, carry):
            for p in range(2):
                grp = g2 * 2 + p
                out_cp = drain_and_out(grp, p)
                out_cp.wait()
                fire(grp + 2, p)
            return carry

        lax.fori_loop(0, (n_grp - 2) // 2, body, 0)

        drain_and_out(n_grp - 2, 0).wait()
        drain_and_out(n_grp - 1, 1).wait()

    out = emb(ids, weight)
    return out.reshape(T, B, D).transpose(1, 0, 2)
